# baseline (speedup 1.0000x reference)
import functools

import jax
import jax.numpy as jnp
from jax import lax
from jax.experimental import pallas as pl
from jax.experimental.pallas import tpu as pltpu
from jax.experimental.pallas import tpu_sc as plsc


def _mlp_body(n_valid, block_rows, blk0, x_ref, ids_ref, w1h_ref, b1h_ref,
              w2_ref, b2_ref, out_ref, ids_out_ref):
    xb = x_ref[...].astype(jnp.bfloat16)
    w1h = w1h_ref[...].astype(jnp.bfloat16)
    b1h = b1h_ref[...]
    u = (lax.dot_general(w1h, xb, (((1,), (1,)), ((), ())),
                         preferred_element_type=jnp.float32)
         + b1h).astype(jnp.bfloat16)
    h = u * jnp.tanh(u) + u
    u = (lax.dot_general(w1h, h, (((1,), (0,)), ((), ())),
                         preferred_element_type=jnp.float32)
         + b1h).astype(jnp.bfloat16)
    h = u * jnp.tanh(u) + u
    w2c = w2_ref[...].astype(jnp.bfloat16)
    pw = (h * w2c).astype(jnp.float32)
    pred = jnp.sum(pw, axis=0) + b2_ref[0, 0]
    base = (pl.program_id(0) + blk0) * block_rows
    gidx = base + lax.broadcasted_iota(jnp.int32, (block_rows,), 0)
    valid = gidx < n_valid
    pred = jnp.where(valid, pred, 0.0)
    ids = jnp.where(valid, ids_ref[...], 0)
    out_ref[0] = pred.reshape(block_rows // _LANE, _LANE)
    ids_out_ref[0] = ids.reshape(block_rows // _LANE, _LANE)


def _mlp_pred(node_embedding, batch, W1, b1, W2, b2, block_rows, blk0,
              nblocks):
    n, d = node_embedding.shape
    body = functools.partial(_mlp_body, n, block_rows, blk0)
    blk3 = (1, block_rows // _LANE, _LANE)
    shp3 = (nblocks, block_rows // _LANE, _LANE)
    return pl.pallas_call(
        body,
        grid=(nblocks,),
        in_specs=[
            pl.BlockSpec((block_rows, d), lambda i: (i + blk0, 0)),
            pl.BlockSpec((block_rows,), lambda i: (i + blk0,)),
            pl.BlockSpec((d, d), lambda i: (0, 0)),
            pl.BlockSpec((d, 1), lambda i: (0, 0)),
            pl.BlockSpec((d, 1), lambda i: (0, 0)),
            pl.BlockSpec((1, 1), lambda i: (0, 0)),
        ],
        out_specs=[pl.BlockSpec(blk3, lambda i: (i, 0, 0)),
                   pl.BlockSpec(blk3, lambda i: (i, 0, 0))],
        out_shape=[jax.ShapeDtypeStruct(shp3, jnp.float32),
                   jax.ShapeDtypeStruct(shp3, jnp.int32)],
    )(node_embedding, batch, 0.5 * W1, (0.5 * b1).reshape(d, 1),
      W2.reshape(d, 1), b2.reshape(1, 1))


_NSUB = 16
_LANE = 128


def _make_seg_sum(nj, nchunks, s_pad, with_init):
    mesh = plsc.VectorSubcoreMesh(core_axis_name="c", subcore_axis_name="s")

    @functools.partial(
        pl.kernel,
        out_type=jax.ShapeDtypeStruct((s_pad,), jnp.float32),
        mesh=mesh,
        scratch_types=[
            pltpu.VMEM((nj, _LANE), jnp.float32),
            pltpu.VMEM((nj, _LANE), jnp.int32),
            pltpu.VMEM((s_pad,), jnp.float32),
            pltpu.VMEM_SHARED((s_pad,), jnp.float32),
            pltpu.SemaphoreType.DMA,
        ],
    )
    def seg_sum(pred_hbm, ids_hbm, *rest):
        if with_init:
            init_hbm, out_hbm, vals_v, ids_v, zero_v, acc_sh, sem = rest
        else:
            out_hbm, vals_v, ids_v, zero_v, acc_sh, sem = rest
            init_hbm = None
        cid = lax.axis_index("c")
        sid = lax.axis_index("s")

        @pl.when(cid == 0)
        def _():
            @pl.when(sid < nchunks)
            def _stage():
                cp_v = pltpu.async_copy(pred_hbm.at[sid], vals_v, sem)
                cp_i = pltpu.async_copy(ids_hbm.at[sid], ids_v, sem)
                cp_v.wait()
                cp_i.wait()

            @pl.when(sid == 0)
            def _acc_init():
                if with_init:
                    pltpu.sync_copy(init_hbm, acc_sh)
                else:
                    @pl.loop(0, s_pad, step=16)
                    def _(i):
                        zero_v[pl.ds(i, 16)] = jnp.zeros((16,), jnp.float32)

                    pltpu.sync_copy(zero_v, acc_sh)

            plsc.subcore_barrier()

            @pl.when(sid < nchunks)
            def _scatter():
                descs = [
                    pltpu.async_copy(vals_v.at[j], acc_sh.at[ids_v.at[j]],
                                     sem, add=True)
                    for j in range(nj)
                ]
                for d_ in descs:
                    d_.wait()

            plsc.subcore_barrier()

            @pl.when(sid == 0)
            def _out():
                pltpu.sync_copy(acc_sh, out_hbm)

    return seg_sum


def kernel(node_embedding, pos, atomic_numbers, batch, natoms, W1, b1, W2, b2):
    n, d = node_embedding.shape
    s = natoms.shape[0]

    s_pad = -(-s // 128) * 128
    nj = 56
    block_rows = nj * _LANE
    nchunks = -(-n // block_rows)
    n_pad = nchunks * block_rows

    ca = nchunks // 2
    cb = nchunks - ca
    pa, ia = _mlp_pred(node_embedding, batch, W1, b1, W2, b2, block_rows,
                       0, ca)
    pb, ib = _mlp_pred(node_embedding, batch, W1, b1, W2, b2, block_rows,
                       ca, cb)
    part = _make_seg_sum(nj, ca, s_pad, False)(pa, ia)
    energy = _make_seg_sum(nj, cb, s_pad, True)(pb, ib, part)
    return energy[:s]


# speedup vs baseline: 1.0537x; 1.0537x over previous
import functools

import jax
import jax.numpy as jnp
from jax import lax
from jax.experimental import pallas as pl
from jax.experimental.pallas import tpu as pltpu
from jax.experimental.pallas import tpu_sc as plsc


def _mlp_body(n_valid, block_rows, blk0, x_ref, ids_ref, w1h_ref, b1h_ref,
              w2_ref, b2_ref, out_ref, ids_out_ref):
    xb = x_ref[...].astype(jnp.bfloat16)
    w1h = w1h_ref[...].astype(jnp.bfloat16)
    b1h = b1h_ref[...]
    u = (lax.dot_general(w1h, xb, (((1,), (1,)), ((), ())),
                         preferred_element_type=jnp.float32)
         + b1h).astype(jnp.bfloat16)
    h = u * jnp.tanh(u) + u
    u = (lax.dot_general(w1h, h, (((1,), (0,)), ((), ())),
                         preferred_element_type=jnp.float32)
         + b1h).astype(jnp.bfloat16)
    h = u * jnp.tanh(u) + u
    w2c = w2_ref[...].astype(jnp.bfloat16)
    pw = (h * w2c).astype(jnp.float32)
    pred = jnp.sum(pw, axis=0) + b2_ref[0, 0]
    base = (pl.program_id(0) + blk0) * block_rows
    gidx = base + lax.broadcasted_iota(jnp.int32, (block_rows,), 0)
    valid = gidx < n_valid
    pred = jnp.where(valid, pred, 0.0)
    ids = jnp.where(valid, ids_ref[...], 0)
    out_ref[0] = pred.reshape(block_rows // _LANE, _LANE)
    ids_out_ref[0] = ids.reshape(block_rows // _LANE, _LANE)


def _mlp_pred(node_embedding, batch, W1, b1, W2, b2, block_rows, blk0,
              nblocks):
    n, d = node_embedding.shape
    body = functools.partial(_mlp_body, n, block_rows, blk0)
    blk3 = (1, block_rows // _LANE, _LANE)
    shp3 = (nblocks, block_rows // _LANE, _LANE)
    return pl.pallas_call(
        body,
        grid=(nblocks,),
        in_specs=[
            pl.BlockSpec((block_rows, d), lambda i: (i + blk0, 0)),
            pl.BlockSpec((block_rows,), lambda i: (i + blk0,)),
            pl.BlockSpec((d, d), lambda i: (0, 0)),
            pl.BlockSpec((d, 1), lambda i: (0, 0)),
            pl.BlockSpec((d, 1), lambda i: (0, 0)),
            pl.BlockSpec((1, 1), lambda i: (0, 0)),
        ],
        out_specs=[pl.BlockSpec(blk3, lambda i: (i, 0, 0)),
                   pl.BlockSpec(blk3, lambda i: (i, 0, 0))],
        out_shape=[jax.ShapeDtypeStruct(shp3, jnp.float32),
                   jax.ShapeDtypeStruct(shp3, jnp.int32)],
    )(node_embedding, batch, 0.5 * W1, (0.5 * b1).reshape(d, 1),
      W2.reshape(d, 1), b2.reshape(1, 1))


_NSUB = 16
_LANE = 128


def _make_seg_sum(nj, nchunks, s_pad, with_init):
    mesh = plsc.VectorSubcoreMesh(core_axis_name="c", subcore_axis_name="s")

    @functools.partial(
        pl.kernel,
        out_type=jax.ShapeDtypeStruct((s_pad,), jnp.float32),
        mesh=mesh,
        scratch_types=[
            pltpu.VMEM((nj, _LANE), jnp.float32),
            pltpu.VMEM((nj, _LANE), jnp.int32),
            pltpu.VMEM((s_pad,), jnp.float32),
            pltpu.VMEM_SHARED((s_pad,), jnp.float32),
            pltpu.SemaphoreType.DMA,
        ],
    )
    def seg_sum(pred_hbm, ids_hbm, *rest):
        if with_init:
            init_hbm, out_hbm, vals_v, ids_v, zero_v, acc_sh, sem = rest
        else:
            out_hbm, vals_v, ids_v, zero_v, acc_sh, sem = rest
            init_hbm = None
        cid = lax.axis_index("c")
        sid = lax.axis_index("s")

        @pl.when(cid == 0)
        def _():
            @pl.when(sid < nchunks)
            def _stage():
                cp_v = pltpu.async_copy(pred_hbm.at[sid], vals_v, sem)
                cp_i = pltpu.async_copy(ids_hbm.at[sid], ids_v, sem)
                cp_v.wait()
                cp_i.wait()

            @pl.when(sid == 0)
            def _acc_init():
                if with_init:
                    pltpu.sync_copy(init_hbm, acc_sh)
                else:
                    @pl.loop(0, s_pad, step=16)
                    def _(i):
                        zero_v[pl.ds(i, 16)] = jnp.zeros((16,), jnp.float32)

                    pltpu.sync_copy(zero_v, acc_sh)

            plsc.subcore_barrier()

            @pl.when(sid < nchunks)
            def _scatter():
                descs = [
                    pltpu.async_copy(vals_v.at[j], acc_sh.at[ids_v.at[j]],
                                     sem, add=True)
                    for j in range(nj)
                ]
                for d_ in descs:
                    d_.wait()

            plsc.subcore_barrier()

            @pl.when(sid == 0)
            def _out():
                pltpu.sync_copy(acc_sh, out_hbm)

    return seg_sum


def kernel(node_embedding, pos, atomic_numbers, batch, natoms, W1, b1, W2, b2):
    n, d = node_embedding.shape
    s = natoms.shape[0]

    s_pad = -(-s // 128) * 128
    nj = 56
    block_rows = nj * _LANE
    nchunks = -(-n // block_rows)
    n_pad = nchunks * block_rows

    pred3, ids3 = _mlp_pred(node_embedding, batch, W1, b1, W2, b2, block_rows,
                            0, nchunks)
    energy = _make_seg_sum(nj, nchunks, s_pad, False)(pred3, ids3)
    return energy[:s]
